# Initial kernel scaffold; baseline (speedup 1.0000x reference)
import jax
import jax.numpy as jnp
from jax.experimental import pallas as pl


def kernel(x, edge_index, edge_index_2bridge, W, att_src, att_dst, bias, gamma, beta):
    raise NotImplementedError("write your pallas kernel here")


# speedup vs baseline: 29.4844x; 29.4844x over previous
import functools

import jax
import jax.numpy as jnp
from jax import lax
from jax.experimental import pallas as pl
from jax.experimental.pallas import tpu as pltpu
from jax.experimental.pallas import tpu_sc as plsc

N = 10000
C = 128
NP = 10240
BLK = 1024
NB = NP // BLK
NCORE = 2
NSUB = 16
NW = NCORE * NSUB
DPT = NP // NSUB
CHUNK = 128

_f32 = jnp.float32
_i32 = jnp.int32


def _lrelu(t):
    return jnp.where(t > 0, t, 0.2 * t)


def _emit_projection(xn, w_ref, asw_ref, adw_ref, h_ref, as_ref, ad_ref,
                     gm_ref, gmacc):
    i = pl.program_id(0)
    h = jnp.dot(xn, w_ref[...], preferred_element_type=_f32)
    h_ref[...] = h
    a_s = jnp.sum(h * asw_ref[...], axis=1, keepdims=True)
    a_d = jnp.sum(h * adw_ref[...], axis=1, keepdims=True)
    as_ref[...] = a_s
    ad_ref[...] = a_d
    m = jnp.max(a_s)

    @pl.when(i == 0)
    def _():
        gmacc[0, 0] = m

    @pl.when(i > 0)
    def _():
        gmacc[0, 0] = jnp.maximum(gmacc[0, 0], m)

    @pl.when(i == pl.num_programs(0) - 1)
    def _():
        gm_ref[...] = jnp.full((8, 128), gmacc[0, 0], _f32)


def _proj_body(x_ref, w_ref, asw_ref, adw_ref, h_ref, as_ref, ad_ref,
               gm_ref, gmacc):
    _emit_projection(x_ref[...], w_ref, asw_ref, adw_ref, h_ref, as_ref,
                     ad_ref, gm_ref, gmacc)


_ROWSPEC = pl.BlockSpec((BLK, C), lambda i: (i, 0))
_WSPEC = pl.BlockSpec((C, C), lambda i: (0, 0))
_VECSPEC = pl.BlockSpec((1, C), lambda i: (0, 0))
_COLSPEC = pl.BlockSpec((BLK, 1), lambda i: (i, 0))
_GMSPEC = pl.BlockSpec((8, 128), lambda i: (0, 0))


def _project(x_p, w, asw, adw):
    return pl.pallas_call(
        _proj_body,
        grid=(NB,),
        in_specs=[_ROWSPEC, _WSPEC, _VECSPEC, _VECSPEC],
        out_specs=[_ROWSPEC, _COLSPEC, _COLSPEC, _GMSPEC],
        out_shape=[
            jax.ShapeDtypeStruct((NP, C), _f32),
            jax.ShapeDtypeStruct((NP, 1), _f32),
            jax.ShapeDtypeStruct((NP, 1), _f32),
            jax.ShapeDtypeStruct((8, 128), _f32),
        ],
        scratch_shapes=[pltpu.SMEM((1, 1), _f32)],
    )(x_p, w, asw, adw)


def _c1_body(p_ref, den_ref, exs_ref, h_ref, og_ref, st_ref, sacc):
    i = pl.program_id(0)
    p = p_ref[0] + p_ref[1]
    den = den_ref[0] + den_ref[1]
    exs = exs_ref[...]
    h = h_ref[...]
    g = (p + exs * h) / (den + 1e-16)
    og_ref[...] = g
    row = i * BLK + lax.broadcasted_iota(_i32, (BLK, 1), 0)
    gm = jnp.where(row < N, g, 0.0)
    s1 = jnp.sum(gm, axis=0, keepdims=True)
    s2 = jnp.sum(gm * gm, axis=0, keepdims=True)

    @pl.when(i == 0)
    def _():
        sacc[0:1, :] = s1
        sacc[1:2, :] = s2

    @pl.when(i > 0)
    def _():
        sacc[0:1, :] = sacc[0:1, :] + s1
        sacc[1:2, :] = sacc[1:2, :] + s2

    @pl.when(i == pl.num_programs(0) - 1)
    def _():
        mean = sacc[0:1, :] * (1.0 / N)
        var = sacc[1:2, :] * (1.0 / N) - mean * mean
        istd = lax.rsqrt(jnp.maximum(var, 0.0) + 1e-5)
        st_ref[0:1, :] = mean
        st_ref[1:2, :] = istd


def _c1(acc, den2, exs2, h):
    return pl.pallas_call(
        _c1_body,
        grid=(NB,),
        in_specs=[
            pl.BlockSpec((2, BLK, C), lambda i: (0, i, 0)),
            pl.BlockSpec((2, BLK, 1), lambda i: (0, i, 0)),
            _COLSPEC,
            _ROWSPEC,
        ],
        out_specs=[_ROWSPEC, _GMSPEC],
        out_shape=[
            jax.ShapeDtypeStruct((NP, C), _f32),
            jax.ShapeDtypeStruct((8, 128), _f32),
        ],
        scratch_shapes=[pltpu.VMEM((8, 128), _f32)],
    )(acc, den2.reshape(2, NP, 1), exs2.reshape(NP, 1), h)


def _bn_act(og_ref, st_ref, gam_ref, bet_ref, r_ref, act, resid):
    g = og_ref[...]
    mean = st_ref[0:1, :]
    istd = st_ref[1:2, :]
    xn = gam_ref[...] * (g - mean) * istd + bet_ref[...]
    if act == "elu":
        xn = jnp.where(xn > 0, xn, jnp.exp(jnp.minimum(xn, 0.0)) - 1.0)
    else:
        xn = jnp.maximum(xn, 0.0)
    if resid:
        xn = jnp.maximum(xn + r_ref[...], 0.0)
    return xn


def _c2_body(og_ref, st_ref, gam_ref, bet_ref, r_ref, w_ref, asw_ref, adw_ref,
             x_ref, h_ref, as_ref, ad_ref, gm_ref, gmacc, *, act, resid):
    xn = _bn_act(og_ref, st_ref, gam_ref, bet_ref, r_ref, act, resid)
    x_ref[...] = xn
    _emit_projection(xn, w_ref, asw_ref, adw_ref, h_ref, as_ref, ad_ref,
                     gm_ref, gmacc)


def _c2(og, st, gam, bet, r, w, asw, adw, act, resid):
    return pl.pallas_call(
        functools.partial(_c2_body, act=act, resid=resid),
        grid=(NB,),
        in_specs=[_ROWSPEC, _GMSPEC, _VECSPEC, _VECSPEC, _ROWSPEC,
                  _WSPEC, _VECSPEC, _VECSPEC],
        out_specs=[_ROWSPEC, _ROWSPEC, _COLSPEC, _COLSPEC, _GMSPEC],
        out_shape=[
            jax.ShapeDtypeStruct((NP, C), _f32),
            jax.ShapeDtypeStruct((NP, C), _f32),
            jax.ShapeDtypeStruct((NP, 1), _f32),
            jax.ShapeDtypeStruct((NP, 1), _f32),
            jax.ShapeDtypeStruct((8, 128), _f32),
        ],
        scratch_shapes=[pltpu.SMEM((1, 1), _f32)],
    )(og, st, gam, bet, r, w, asw, adw)


def _c2f_body(og_ref, st_ref, gam_ref, bet_ref, r_ref, x_ref, *, act, resid):
    x_ref[...] = _bn_act(og_ref, st_ref, gam_ref, bet_ref, r_ref, act, resid)


def _c2_final(og, st, gam, bet, r, act, resid):
    return pl.pallas_call(
        functools.partial(_c2f_body, act=act, resid=resid),
        grid=(NB,),
        in_specs=[_ROWSPEC, _GMSPEC, _VECSPEC, _VECSPEC, _ROWSPEC],
        out_specs=_ROWSPEC,
        out_shape=jax.ShapeDtypeStruct((NP, C), _f32),
    )(og, st, gam, bet, r)


def _sc_edge_pass(h, a_s, a_d, gm16, e_src, e_dst):
    e_total = e_src.shape[0]
    epw = e_total // NW
    nfull = epw // CHUNK
    rem = epw - nfull * CHUNK

    mesh = plsc.VectorSubcoreMesh(core_axis_name="c", subcore_axis_name="s",
                                  num_cores=NCORE, num_subcores=NSUB)

    def body(h_hbm, as_hbm, ad_hbm, gm_hbm, es_hbm, ed_hbm,
             acc_out, den_out, self_out,
             asv, adv, gmv, srcv, dstv, rows, exv,
             srcv2, dstv2, rows2, exv2, selfb, initb, zblk,
             acc_s, den_s, sem):
        cid = lax.axis_index("c")
        sid = lax.axis_index("s")
        wid = sid * NCORE + cid

        pltpu.sync_copy(as_hbm, asv)
        pltpu.sync_copy(ad_hbm, adv)
        pltpu.sync_copy(gm_hbm, gmv)
        gmax = gmv[...]
        z16 = jnp.zeros((16,), _f32)
        for r in range(16):
            for cc in range(8):
                zblk[r, pl.ds(cc * 16, 16)] = z16

        for g in range(DPT // 16):
            j0 = sid * DPT + g * 16
            asg = asv[pl.ds(j0, 16)]
            adg = adv[pl.ds(j0, 16)]
            al = _lrelu(asg + adg)
            cl = _lrelu(adg + gmax)
            exg = jnp.exp(al - cl)
            selfb[pl.ds(g * 16, 16)] = exg
            initb[pl.ds(g * 16, 16)] = jnp.where(cid == 0, exg, 0.0)

        pltpu.sync_copy(initb, den_s.at[pl.ds(sid * DPT, DPT)])

        @pl.when(cid == 0)
        def _():
            pltpu.sync_copy(selfb, self_out.at[pl.ds(sid * DPT, DPT)])

        for b in range(DPT // 16):
            pltpu.sync_copy(zblk, acc_s.at[pl.ds(sid * DPT + b * 16, 16), :])

        plsc.subcore_barrier()

        def compute_chunk(src_ref, dst_ref, rows_ref, ex_ref, k):
            ngrp = k // 16
            for g in range(ngrp):
                sidx = src_ref[pl.ds(g * 16, 16)]
                didx = dst_ref[pl.ds(g * 16, 16)]
                asg = plsc.load_gather(asv, [sidx])
                adg = plsc.load_gather(adv, [didx])
                al = _lrelu(asg + adg)
                cl = _lrelu(adg + gmax)
                exg = jnp.exp(al - cl)
                ex_ref[pl.ds(g * 16, 16)] = exg
            pltpu.sync_copy(ex_ref, den_s.at[dst_ref], add=True)
            for g in range(ngrp):
                exgrp = ex_ref[pl.ds(g * 16, 16)]
                for r2 in range(16):
                    r = g * 16 + r2
                    exb = exgrp.at[jnp.full((16,), r2, _i32)].get(
                        mode="promise_in_bounds")
                    for cc in range(8):
                        rows_ref[r, pl.ds(cc * 16, 16)] = (
                            rows_ref[r, pl.ds(cc * 16, 16)] * exb)
            pltpu.sync_copy(rows_ref, acc_s.at[dst_ref], add=True)

        ebase = wid * epw

        def chunk_body(ci, carry):
            base = ebase + ci * CHUNK
            pltpu.sync_copy(es_hbm.at[pl.ds(base, CHUNK)], srcv)
            pltpu.sync_copy(ed_hbm.at[pl.ds(base, CHUNK)], dstv)
            pltpu.async_copy(h_hbm.at[srcv], rows, sem).wait()
            compute_chunk(srcv, dstv, rows, exv, CHUNK)
            return carry

        lax.fori_loop(0, nfull, chunk_body, 0)

        if rem:
            base = ebase + nfull * CHUNK
            pltpu.sync_copy(es_hbm.at[pl.ds(base, rem)], srcv2)
            pltpu.sync_copy(ed_hbm.at[pl.ds(base, rem)], dstv2)
            pltpu.async_copy(h_hbm.at[srcv2], rows2, sem).wait()
            compute_chunk(srcv2, dstv2, rows2, exv2, rem)

        plsc.subcore_barrier()

        pltpu.sync_copy(acc_s.at[pl.ds(sid * DPT, DPT), :],
                        acc_out.at[cid, pl.ds(sid * DPT, DPT), :])
        pltpu.sync_copy(den_s.at[pl.ds(sid * DPT, DPT)],
                        den_out.at[cid, pl.ds(sid * DPT, DPT)])

    def wrapped(h_, as_, ad_, gm_, es_, ed_):
        return pl.kernel(
            body,
            out_type=(
                jax.ShapeDtypeStruct((NCORE, NP, C), _f32),
                jax.ShapeDtypeStruct((NCORE, NP), _f32),
                jax.ShapeDtypeStruct((NP,), _f32),
            ),
            mesh=mesh,
            compiler_params=pltpu.CompilerParams(needs_layout_passes=False),
            scratch_types=[
                pltpu.VMEM((NP,), _f32),
                pltpu.VMEM((NP,), _f32),
                pltpu.VMEM((16,), _f32),
                pltpu.VMEM((CHUNK,), _i32),
                pltpu.VMEM((CHUNK,), _i32),
                pltpu.VMEM((CHUNK, C), _f32),
                pltpu.VMEM((CHUNK,), _f32),
                pltpu.VMEM((rem or 16,), _i32),
                pltpu.VMEM((rem or 16,), _i32),
                pltpu.VMEM((rem or 16, C), _f32),
                pltpu.VMEM((rem or 16,), _f32),
                pltpu.VMEM((DPT,), _f32),
                pltpu.VMEM((DPT,), _f32),
                pltpu.VMEM((16, C), _f32),
                pltpu.VMEM_SHARED((NP, C), _f32),
                pltpu.VMEM_SHARED((NP,), _f32),
                pltpu.SemaphoreType.DMA,
            ],
        )(h_, as_, ad_, gm_, es_, ed_)

    return wrapped(h, a_s, a_d, gm16, e_src, e_dst)


def kernel(x, edge_index, edge_index_2bridge, W, att_src, att_dst, bias,
           gamma, beta):
    del bias
    x_p = jnp.pad(x.astype(_f32), ((0, NP - N), (0, 0)))
    edge_lists = [edge_index, edge_index, edge_index, edge_index_2bridge,
                  edge_index, edge_index_2bridge]
    acts = ["elu", "relu", "elu", "relu", "elu", "relu"]

    h, as2, ad2, gm = _project(x_p, W[0], att_src[0].reshape(1, C),
                               att_dst[0].reshape(1, C))
    resid = x_p
    x_cur = x_p
    for i in range(6):
        a_s = as2.reshape(NP)
        a_d = ad2.reshape(NP)
        gm16 = gm.reshape(-1)[:16]
        ed = edge_lists[i].astype(_i32)
        acc, den, exs = _sc_edge_pass(h, a_s, a_d, gm16, ed[0], ed[1])
        og, st = _c1(acc, den, exs, h)
        use_res = i in (3, 5)
        gam = gamma[i].reshape(1, C)
        bet = beta[i].reshape(1, C)
        if i == 2:
            resid = x_cur
        if i == 4:
            resid = x_cur
        if i < 5:
            x_cur, h, as2, ad2, gm = _c2(
                og, st, gam, bet, resid, W[i + 1],
                att_src[i + 1].reshape(1, C), att_dst[i + 1].reshape(1, C),
                acts[i], use_res)
        else:
            x_cur = _c2_final(og, st, gam, bet, resid, acts[i], use_res)
    return x_cur[:N]


# speedup vs baseline: 31.4873x; 1.0679x over previous
import functools

import jax
import jax.numpy as jnp
from jax import lax
from jax.experimental import pallas as pl
from jax.experimental.pallas import tpu as pltpu
from jax.experimental.pallas import tpu_sc as plsc

N = 10000
C = 128
NP = 10240
BLK = 1024
NB = NP // BLK
NCORE = 2
NSUB = 16
NW = NCORE * NSUB
DPT = NP // NSUB
CHUNK = 128

_f32 = jnp.float32
_i32 = jnp.int32


def _lrelu(t):
    return jnp.where(t > 0, t, 0.2 * t)


def _emit_projection(xn, w_ref, asw_ref, adw_ref, h_ref, as_ref, ad_ref,
                     gm_ref, gmacc):
    i = pl.program_id(0)
    h = jnp.dot(xn, w_ref[...], preferred_element_type=_f32)
    h_ref[...] = h
    a_s = jnp.sum(h * asw_ref[...], axis=1, keepdims=True)
    a_d = jnp.sum(h * adw_ref[...], axis=1, keepdims=True)
    as_ref[...] = a_s
    ad_ref[...] = a_d
    m = jnp.max(a_s)

    @pl.when(i == 0)
    def _():
        gmacc[0, 0] = m

    @pl.when(i > 0)
    def _():
        gmacc[0, 0] = jnp.maximum(gmacc[0, 0], m)

    @pl.when(i == pl.num_programs(0) - 1)
    def _():
        gm_ref[...] = jnp.full((8, 128), gmacc[0, 0], _f32)


def _proj_body(x_ref, w_ref, asw_ref, adw_ref, h_ref, as_ref, ad_ref,
               gm_ref, gmacc):
    _emit_projection(x_ref[...], w_ref, asw_ref, adw_ref, h_ref, as_ref,
                     ad_ref, gm_ref, gmacc)


_ROWSPEC = pl.BlockSpec((BLK, C), lambda i: (i, 0))
_WSPEC = pl.BlockSpec((C, C), lambda i: (0, 0))
_VECSPEC = pl.BlockSpec((1, C), lambda i: (0, 0))
_COLSPEC = pl.BlockSpec((BLK, 1), lambda i: (i, 0))
_GMSPEC = pl.BlockSpec((8, 128), lambda i: (0, 0))


def _project(x_p, w, asw, adw):
    return pl.pallas_call(
        _proj_body,
        grid=(NB,),
        in_specs=[_ROWSPEC, _WSPEC, _VECSPEC, _VECSPEC],
        out_specs=[_ROWSPEC, _COLSPEC, _COLSPEC, _GMSPEC],
        out_shape=[
            jax.ShapeDtypeStruct((NP, C), _f32),
            jax.ShapeDtypeStruct((NP, 1), _f32),
            jax.ShapeDtypeStruct((NP, 1), _f32),
            jax.ShapeDtypeStruct((8, 128), _f32),
        ],
        scratch_shapes=[pltpu.SMEM((1, 1), _f32)],
    )(x_p, w, asw, adw)


def _c1_body(p_ref, den_ref, exs_ref, h_ref, og_ref, st_ref, sacc):
    i = pl.program_id(0)
    p = p_ref[0] + p_ref[1]
    den = den_ref[0] + den_ref[1]
    exs = exs_ref[...]
    h = h_ref[...]
    g = (p + exs * h) / (den + 1e-16)
    og_ref[...] = g
    row = i * BLK + lax.broadcasted_iota(_i32, (BLK, 1), 0)
    gm = jnp.where(row < N, g, 0.0)
    s1 = jnp.sum(gm, axis=0, keepdims=True)
    s2 = jnp.sum(gm * gm, axis=0, keepdims=True)

    @pl.when(i == 0)
    def _():
        sacc[0:1, :] = s1
        sacc[1:2, :] = s2

    @pl.when(i > 0)
    def _():
        sacc[0:1, :] = sacc[0:1, :] + s1
        sacc[1:2, :] = sacc[1:2, :] + s2

    @pl.when(i == pl.num_programs(0) - 1)
    def _():
        mean = sacc[0:1, :] * (1.0 / N)
        var = sacc[1:2, :] * (1.0 / N) - mean * mean
        istd = lax.rsqrt(jnp.maximum(var, 0.0) + 1e-5)
        st_ref[0:1, :] = mean
        st_ref[1:2, :] = istd


def _c1(acc, den2, exs2, h):
    return pl.pallas_call(
        _c1_body,
        grid=(NB,),
        in_specs=[
            pl.BlockSpec((2, BLK, C), lambda i: (0, i, 0)),
            pl.BlockSpec((2, BLK, 1), lambda i: (0, i, 0)),
            _COLSPEC,
            _ROWSPEC,
        ],
        out_specs=[_ROWSPEC, _GMSPEC],
        out_shape=[
            jax.ShapeDtypeStruct((NP, C), _f32),
            jax.ShapeDtypeStruct((8, 128), _f32),
        ],
        scratch_shapes=[pltpu.VMEM((8, 128), _f32)],
    )(acc, den2.reshape(2, NP, 1), exs2.reshape(NP, 1), h)


def _bn_act(og_ref, st_ref, gam_ref, bet_ref, r_ref, act, resid):
    g = og_ref[...]
    mean = st_ref[0:1, :]
    istd = st_ref[1:2, :]
    xn = gam_ref[...] * (g - mean) * istd + bet_ref[...]
    if act == "elu":
        xn = jnp.where(xn > 0, xn, jnp.exp(jnp.minimum(xn, 0.0)) - 1.0)
    else:
        xn = jnp.maximum(xn, 0.0)
    if resid:
        xn = jnp.maximum(xn + r_ref[...], 0.0)
    return xn


def _c2_body(og_ref, st_ref, gam_ref, bet_ref, r_ref, w_ref, asw_ref, adw_ref,
             x_ref, h_ref, as_ref, ad_ref, gm_ref, gmacc, *, act, resid):
    xn = _bn_act(og_ref, st_ref, gam_ref, bet_ref, r_ref, act, resid)
    x_ref[...] = xn
    _emit_projection(xn, w_ref, asw_ref, adw_ref, h_ref, as_ref, ad_ref,
                     gm_ref, gmacc)


def _c2(og, st, gam, bet, r, w, asw, adw, act, resid):
    return pl.pallas_call(
        functools.partial(_c2_body, act=act, resid=resid),
        grid=(NB,),
        in_specs=[_ROWSPEC, _GMSPEC, _VECSPEC, _VECSPEC, _ROWSPEC,
                  _WSPEC, _VECSPEC, _VECSPEC],
        out_specs=[_ROWSPEC, _ROWSPEC, _COLSPEC, _COLSPEC, _GMSPEC],
        out_shape=[
            jax.ShapeDtypeStruct((NP, C), _f32),
            jax.ShapeDtypeStruct((NP, C), _f32),
            jax.ShapeDtypeStruct((NP, 1), _f32),
            jax.ShapeDtypeStruct((NP, 1), _f32),
            jax.ShapeDtypeStruct((8, 128), _f32),
        ],
        scratch_shapes=[pltpu.SMEM((1, 1), _f32)],
    )(og, st, gam, bet, r, w, asw, adw)


def _c2f_body(og_ref, st_ref, gam_ref, bet_ref, r_ref, x_ref, *, act, resid):
    x_ref[...] = _bn_act(og_ref, st_ref, gam_ref, bet_ref, r_ref, act, resid)


def _c2_final(og, st, gam, bet, r, act, resid):
    return pl.pallas_call(
        functools.partial(_c2f_body, act=act, resid=resid),
        grid=(NB,),
        in_specs=[_ROWSPEC, _GMSPEC, _VECSPEC, _VECSPEC, _ROWSPEC],
        out_specs=_ROWSPEC,
        out_shape=jax.ShapeDtypeStruct((NP, C), _f32),
    )(og, st, gam, bet, r)


def _sc_edge_pass(h, a_s, a_d, gm16, e_src, e_dst):
    e_total = e_src.shape[0]
    epw = e_total // NW
    nfull = epw // CHUNK
    rem = epw - nfull * CHUNK
    assert nfull % 2 == 0 and rem in (0, 16)

    mesh = plsc.VectorSubcoreMesh(core_axis_name="c", subcore_axis_name="s",
                                  num_cores=NCORE, num_subcores=NSUB)

    def body(h_hbm, as_hbm, ad_hbm, gm_hbm, es_hbm, ed_hbm,
             acc_out, den_out, self_out,
             asv, gmv,
             srcv0, dstv0, rows0, exv0, adx0,
             srcv1, dstv1, rows1, exv1, adx1,
             srcv2, dstv2, exv2, adx2, adl, selfb, initb, zblk,
             acc_s, den_s,
             gsem0, gsem1, sem):
        srcvs = (srcv0, srcv1)
        dstvs = (dstv0, dstv1)
        rowss = (rows0, rows1)
        exvs = (exv0, exv1)
        adxs = (adx0, adx1)
        gsems = (gsem0, gsem1)
        cid = lax.axis_index("c")
        sid = lax.axis_index("s")
        wid = sid * NCORE + cid

        pltpu.sync_copy(as_hbm, asv)
        pltpu.sync_copy(gm_hbm, gmv)
        gmax = gmv[...]
        z16 = jnp.zeros((16,), _f32)
        for r in range(8):
            for cc in range(8):
                zblk[r, pl.ds(cc * 16, 16)] = z16

        pltpu.sync_copy(ad_hbm.at[pl.ds(sid * DPT, DPT)], adl)
        for g in range(DPT // 16):
            asg = asv[pl.ds(sid * DPT + g * 16, 16)]
            adg = adl[pl.ds(g * 16, 16)]
            al = _lrelu(asg + adg)
            cl = _lrelu(adg + gmax)
            exg = jnp.exp(al - cl)
            selfb[pl.ds(g * 16, 16)] = exg
            initb[pl.ds(g * 16, 16)] = jnp.where(cid == 0, exg, 0.0)

        pltpu.sync_copy(initb, den_s.at[pl.ds(sid * DPT, DPT)])

        @pl.when(cid == 0)
        def _():
            pltpu.sync_copy(selfb, self_out.at[pl.ds(sid * DPT, DPT)])

        for b in range(DPT // 8):
            pltpu.sync_copy(zblk, acc_s.at[pl.ds(sid * DPT + b * 8, 8), :])

        plsc.subcore_barrier()

        ebase = wid * epw

        def prefetch(b, chunk_idx, guarded):
            base = ebase + chunk_idx * CHUNK

            def start():
                pltpu.sync_copy(es_hbm.at[pl.ds(base, CHUNK)], srcvs[b])
                pltpu.sync_copy(ed_hbm.at[pl.ds(base, CHUNK)], dstvs[b])
                pltpu.async_copy(h_hbm.at[srcvs[b]], rowss[b], gsems[b])
                pltpu.async_copy(ad_hbm.at[dstvs[b]], adxs[b], gsems[b])

            if guarded:
                @pl.when(chunk_idx < nfull)
                def _():
                    start()
            else:
                start()

        def wait_gathers(b):
            pltpu.make_async_copy(h_hbm.at[srcvs[b]], rowss[b],
                                  gsems[b]).wait()
            pltpu.make_async_copy(ad_hbm.at[dstvs[b]], adxs[b],
                                  gsems[b]).wait()

        def compute_ex(src_ref, adx_ref, ex_ref, k):
            for g in range(k // 16):
                sidx = src_ref[pl.ds(g * 16, 16)]
                asg = plsc.load_gather(asv, [sidx])
                adg = adx_ref[pl.ds(g * 16, 16)]
                al = _lrelu(asg + adg)
                cl = _lrelu(adg + gmax)
                ex_ref[pl.ds(g * 16, 16)] = jnp.exp(al - cl)

        def scale_rows(rows_ref, ex_ref, k):
            for g in range(k // 16):
                exgrp = ex_ref[pl.ds(g * 16, 16)]
                for r2 in range(16):
                    r = g * 16 + r2
                    exb = exgrp.at[jnp.full((16,), r2, _i32)].get(
                        mode="promise_in_bounds")
                    for cc in range(8):
                        rows_ref[r, pl.ds(cc * 16, 16)] = (
                            rows_ref[r, pl.ds(cc * 16, 16)] * exb)

        def process(b):
            wait_gathers(b)
            compute_ex(srcvs[b], adxs[b], exvs[b], CHUNK)
            pltpu.sync_copy(exvs[b], den_s.at[dstvs[b]], add=True)
            scale_rows(rowss[b], exvs[b], CHUNK)
            pltpu.sync_copy(rowss[b], acc_s.at[dstvs[b]], add=True)

        prefetch(0, 0, False)

        def outer(ci, carry):
            k = ci * 2
            for j in range(2):
                prefetch(1 - j, k + j + 1, True)
                process(j)
            return carry

        lax.fori_loop(0, nfull // 2, outer, 0)

        if rem:
            base = ebase + nfull * CHUNK
            pltpu.sync_copy(es_hbm.at[pl.ds(base, rem)], srcv2)
            pltpu.sync_copy(ed_hbm.at[pl.ds(base, rem)], dstv2)
            pltpu.async_copy(h_hbm.at[srcv2], rows0.at[pl.ds(0, rem), :],
                             sem).wait()
            pltpu.async_copy(ad_hbm.at[dstv2], adx2, sem).wait()
            compute_ex(srcv2, adx2, exv2, rem)
            pltpu.sync_copy(exv2, den_s.at[dstv2], add=True)
            scale_rows(rows0, exv2, rem)
            pltpu.sync_copy(rows0.at[pl.ds(0, rem), :], acc_s.at[dstv2],
                            add=True)

        plsc.subcore_barrier()

        pltpu.sync_copy(acc_s.at[pl.ds(sid * DPT, DPT), :],
                        acc_out.at[cid, pl.ds(sid * DPT, DPT), :])
        pltpu.sync_copy(den_s.at[pl.ds(sid * DPT, DPT)],
                        den_out.at[cid, pl.ds(sid * DPT, DPT)])

    def wrapped(h_, as_, ad_, gm_, es_, ed_):
        return pl.kernel(
            body,
            out_type=(
                jax.ShapeDtypeStruct((NCORE, NP, C), _f32),
                jax.ShapeDtypeStruct((NCORE, NP), _f32),
                jax.ShapeDtypeStruct((NP,), _f32),
            ),
            mesh=mesh,
            compiler_params=pltpu.CompilerParams(needs_layout_passes=False),
            scratch_types=(
                [
                    pltpu.VMEM((NP,), _f32),
                    pltpu.VMEM((16,), _f32),
                ]
                + [
                    t
                    for _ in range(2)
                    for t in (
                        pltpu.VMEM((CHUNK,), _i32),
                        pltpu.VMEM((CHUNK,), _i32),
                        pltpu.VMEM((CHUNK, C), _f32),
                        pltpu.VMEM((CHUNK,), _f32),
                        pltpu.VMEM((CHUNK,), _f32),
                    )
                ]
                + [
                    pltpu.VMEM((rem or 16,), _i32),
                    pltpu.VMEM((rem or 16,), _i32),
                    pltpu.VMEM((rem or 16,), _f32),
                    pltpu.VMEM((rem or 16,), _f32),
                    pltpu.VMEM((DPT,), _f32),
                    pltpu.VMEM((DPT,), _f32),
                    pltpu.VMEM((DPT,), _f32),
                    pltpu.VMEM((8, C), _f32),
                    pltpu.VMEM_SHARED((NP, C), _f32),
                    pltpu.VMEM_SHARED((NP,), _f32),
                ]
                + [pltpu.SemaphoreType.DMA] * 3
            ),
        )(h_, as_, ad_, gm_, es_, ed_)

    return wrapped(h, a_s, a_d, gm16, e_src, e_dst)


def kernel(x, edge_index, edge_index_2bridge, W, att_src, att_dst, bias,
           gamma, beta):
    del bias
    x_p = jnp.pad(x.astype(_f32), ((0, NP - N), (0, 0)))
    edge_lists = [edge_index, edge_index, edge_index, edge_index_2bridge,
                  edge_index, edge_index_2bridge]
    acts = ["elu", "relu", "elu", "relu", "elu", "relu"]

    h, as2, ad2, gm = _project(x_p, W[0], att_src[0].reshape(1, C),
                               att_dst[0].reshape(1, C))
    resid = x_p
    x_cur = x_p
    for i in range(6):
        a_s = as2.reshape(NP)
        a_d = ad2.reshape(NP)
        gm16 = gm.reshape(-1)[:16]
        ed = edge_lists[i].astype(_i32)
        acc, den, exs = _sc_edge_pass(h, a_s, a_d, gm16, ed[0], ed[1])
        og, st = _c1(acc, den, exs, h)
        use_res = i in (3, 5)
        gam = gamma[i].reshape(1, C)
        bet = beta[i].reshape(1, C)
        if i == 2:
            resid = x_cur
        if i == 4:
            resid = x_cur
        if i < 5:
            x_cur, h, as2, ad2, gm = _c2(
                og, st, gam, bet, resid, W[i + 1],
                att_src[i + 1].reshape(1, C), att_dst[i + 1].reshape(1, C),
                acts[i], use_res)
        else:
            x_cur = _c2_final(og, st, gam, bet, resid, acts[i], use_res)
    return x_cur[:N]
